# baseline (speedup 1.0000x reference)
import jax
import jax.numpy as jnp
from jax import lax
from jax.experimental import pallas as pl
from jax.experimental.pallas import tpu as pltpu
from jax.experimental.pallas import tpu_sc as plsc

N = 10000
E = 320000
D_IN = 128
D_H = 128
N_CLASSES = 64

NC = 2
NS = 16
NW = NC * NS
EPT = E // NW
CHUNK = 128
NCHUNK = 160
NBUF = 4
E_PAD = NS * NCHUNK * CHUNK
DUMMY = 512
ACC_ROWS = N + DUMMY
ZROWS = 624
RB = 1000

_mesh = plsc.VectorSubcoreMesh(core_axis_name="c", subcore_axis_name="s")
_sc_params = pltpu.CompilerParams(needs_layout_passes=False, use_tc_tiling_on_sc=False)


def _deg_body(src_hbm, dst_hbm, osrc_hbm, odst_hbm, idx_v, hist_v):
    cid = lax.axis_index("c")
    sid = lax.axis_index("s")
    w = cid * NS + sid
    zeros16 = jnp.zeros((16,), jnp.float32)
    ones16 = jnp.ones((16,), jnp.float32)
    for ix_hbm, out_hbm in ((src_hbm, osrc_hbm), (dst_hbm, odst_hbm)):
        pltpu.sync_copy(ix_hbm.at[w], idx_v)

        @pl.loop(0, N // 16)
        def _(i):
            hist_v[pl.ds(i * 16, 16)] = zeros16

        @pl.loop(0, EPT // 16)
        def _(i):
            idx = idx_v[pl.ds(i * 16, 16)]
            plsc.addupdate_scatter(hist_v, [idx], ones16)

        pltpu.sync_copy(hist_v, out_hbm.at[w])


def _sc_degrees(src_flat, dst_flat):
    return pl.kernel(
        _deg_body,
        out_type=(
            jax.ShapeDtypeStruct((NW, N), jnp.float32),
            jax.ShapeDtypeStruct((NW, N), jnp.float32),
        ),
        mesh=_mesh,
        scratch_types=[
            pltpu.VMEM((EPT,), jnp.int32),
            pltpu.VMEM((N,), jnp.float32),
        ],
        compiler_params=_sc_params,
    )(src_flat, dst_flat)


def _make_agg_body(dh):
    def _agg_body(hw_hbm, src_hbm, dst_hbm, zeros_hbm, out_hbm,
                  src_v, dst_v, *rest):
        bufs = rest[:NBUF]
        acc_sh = rest[NBUF]
        gsems = rest[NBUF + 1:2 * NBUF + 1]
        ssems = rest[2 * NBUF + 1:3 * NBUF + 1]
        cid = lax.axis_index("c")
        sid = lax.axis_index("s")
        hw_c = hw_hbm.at[cid]
        pltpu.sync_copy(src_hbm.at[sid], src_v)
        pltpu.sync_copy(dst_hbm.at[sid], dst_v)
        pltpu.sync_copy(zeros_hbm, acc_sh.at[pl.ds(sid * ZROWS, ZROWS)])

        @pl.when(sid == NS - 1)
        def _():
            pltpu.sync_copy(zeros_hbm.at[pl.ds(0, ACC_ROWS - NS * ZROWS)],
                            acc_sh.at[pl.ds(NS * ZROWS, ACC_ROWS - NS * ZROWS)])

        plsc.subcore_barrier()

        for u in range(3):
            pltpu.async_copy(hw_c.at[src_v.at[u]], bufs[u], gsems[u])

        @pl.loop(0, NCHUNK, step=NBUF)
        def _(j):
            for u in range(NBUF):
                jj = j + u
                w = (u + 3) % NBUF
                pltpu.make_async_copy(hw_c.at[src_v.at[jj]],
                                      bufs[u], gsems[u]).wait()

                @pl.when(jj + 3 < NCHUNK)
                def _(w=w, jj=jj):
                    pltpu.async_copy(hw_c.at[src_v.at[jj + 3]],
                                     bufs[w], gsems[w])

                pltpu.sync_copy(bufs[u], acc_sh.at[dst_v.at[jj]], add=True)

        plsc.subcore_barrier()

        @pl.loop(0, ZROWS // 104)
        def _(k):
            r0 = sid * ZROWS + k * 104
            pltpu.sync_copy(acc_sh.at[pl.ds(r0, 104)], bufs[0].at[pl.ds(0, 104)])
            pltpu.sync_copy(bufs[0].at[pl.ds(0, 104)],
                            out_hbm.at[cid, pl.ds(r0, 104)])

        @pl.when(sid == NS - 1)
        def _():
            r0 = NS * ZROWS
            pltpu.sync_copy(acc_sh.at[pl.ds(r0, N - r0)],
                            bufs[0].at[pl.ds(0, N - r0)])
            pltpu.sync_copy(bufs[0].at[pl.ds(0, N - r0)],
                            out_hbm.at[cid, pl.ds(r0, N - r0)])

    return _agg_body


def _sc_aggregate(hw, src_t, dst_t, zeros_blk, dh):
    return pl.kernel(
        _make_agg_body(dh),
        out_type=jax.ShapeDtypeStruct((NC, N, dh), jnp.float32),
        mesh=_mesh,
        scratch_types=[
            pltpu.VMEM((NCHUNK, CHUNK), jnp.int32),
            pltpu.VMEM((NCHUNK, CHUNK), jnp.int32),
        ] + [pltpu.VMEM((CHUNK, dh), jnp.float32) for _ in range(NBUF)]
        + [pltpu.VMEM_SHARED((ACC_ROWS, dh), jnp.float32)]
        + [pltpu.SemaphoreType.DMA for _ in range(2 * NBUF)],
        compiler_params=_sc_params,
    )(hw, src_t, dst_t, zeros_blk)


def _mm0_body(x_ref, w_ref, o_ref):
    h = jnp.maximum(x_ref[...], 0.0)
    o_ref[...] = jnp.dot(h, w_ref[...], preferred_element_type=jnp.float32)


def _tc_mm0(x, w0):
    return pl.pallas_call(
        _mm0_body,
        grid=(N // RB,),
        in_specs=[
            pl.BlockSpec((RB, D_IN), lambda i: (i, 0)),
            pl.BlockSpec((D_IN, D_H), lambda i: (0, 0)),
        ],
        out_specs=pl.BlockSpec((RB, D_H), lambda i: (i, 0)),
        out_shape=jax.ShapeDtypeStruct((N, D_H), jnp.float32),
    )(x, w0)


def _norms_body(hs_ref, hd_ref, t_ref, ns_ref, nd_ref, hw_ref):
    ns = lax.rsqrt(jnp.clip(jnp.sum(hs_ref[...], axis=0), 1.0, None))
    nd = lax.rsqrt(jnp.clip(jnp.sum(hd_ref[...], axis=0), 1.0, None))
    ns_ref[...] = ns[:, None]
    nd_ref[...] = nd[:, None]
    hw = t_ref[...] * ns[:, None]
    hw_ref[0] = hw[:, : D_H // 2]
    hw_ref[1] = hw[:, D_H // 2:]


def _tc_norms(hist_src, hist_dst, t0):
    return pl.pallas_call(
        _norms_body,
        out_shape=[
            jax.ShapeDtypeStruct((N, 1), jnp.float32),
            jax.ShapeDtypeStruct((N, 1), jnp.float32),
            jax.ShapeDtypeStruct((NC, N, D_H // 2), jnp.float32),
        ],
    )(hist_src, hist_dst, t0)


def _layer_body(p_ref, nd_ref, ns_ref, b_ref, w_ref, o_ref):
    agg = jnp.concatenate([p_ref[0], p_ref[1]], axis=1)
    h = jnp.maximum(agg * nd_ref[...] + b_ref[...], 0.0)
    hw = jnp.dot(h * ns_ref[...], w_ref[...],
                 preferred_element_type=jnp.float32)
    dh = hw.shape[-1] // 2
    o_ref[0] = hw[:, :dh]
    o_ref[1] = hw[:, dh:]


def _tc_layer(p, norm_dst, norm_src, b, w, d_out):
    d_in_h = p.shape[-1]
    return pl.pallas_call(
        _layer_body,
        grid=(N // RB,),
        in_specs=[
            pl.BlockSpec((NC, RB, d_in_h), lambda i: (0, i, 0)),
            pl.BlockSpec((RB, 1), lambda i: (i, 0)),
            pl.BlockSpec((RB, 1), lambda i: (i, 0)),
            pl.BlockSpec((1, 2 * d_in_h), lambda i: (0, 0)),
            pl.BlockSpec((2 * d_in_h, d_out), lambda i: (0, 0)),
        ],
        out_specs=pl.BlockSpec((NC, RB, d_out // 2), lambda i: (0, i, 0)),
        out_shape=jax.ShapeDtypeStruct((NC, N, d_out // 2), jnp.float32),
    )(p, norm_dst, norm_src, b, w)


def _final_body(p_ref, nd_ref, b_ref, o_ref):
    agg = jnp.concatenate([p_ref[0], p_ref[1]], axis=1)
    o_ref[...] = agg * nd_ref[...] + b_ref[...]


def _tc_final(p, norm_dst, b):
    dh = p.shape[-1]
    return pl.pallas_call(
        _final_body,
        grid=(N // RB,),
        in_specs=[
            pl.BlockSpec((NC, RB, dh), lambda i: (0, i, 0)),
            pl.BlockSpec((RB, 1), lambda i: (i, 0)),
            pl.BlockSpec((1, 2 * dh), lambda i: (0, 0)),
        ],
        out_specs=pl.BlockSpec((RB, 2 * dh), lambda i: (i, 0)),
        out_shape=jax.ShapeDtypeStruct((N, 2 * dh), jnp.float32),
    )(p, norm_dst, b)


def kernel(x, edge_index, W0, b0, W1, b1, W2, b2):
    src = edge_index[0]
    dst = edge_index[1]
    src_flat = src.reshape(NW, EPT)
    dst_flat = dst.reshape(NW, EPT)
    pad = E_PAD - E
    src_t = jnp.concatenate(
        [src, jnp.zeros((pad,), jnp.int32)]).reshape(NS, NCHUNK, CHUNK)
    dst_t = jnp.concatenate(
        [dst, N + (jnp.arange(pad, dtype=jnp.int32) % DUMMY)]
    ).reshape(NS, NCHUNK, CHUNK)
    zeros64 = jnp.zeros((ZROWS, D_H // 2), jnp.float32)
    zeros32 = jnp.zeros((ZROWS, N_CLASSES // 2), jnp.float32)

    hist_src, hist_dst = _sc_degrees(src_flat, dst_flat)
    t0 = _tc_mm0(x, W0)
    norm_src, norm_dst, hw0 = _tc_norms(hist_src, hist_dst, t0)

    p0 = _sc_aggregate(hw0, src_t, dst_t, zeros64, D_H // 2)
    hw1 = _tc_layer(p0, norm_dst, norm_src, b0.reshape(1, D_H), W1, D_H)
    p1 = _sc_aggregate(hw1, src_t, dst_t, zeros64, D_H // 2)
    hw2 = _tc_layer(p1, norm_dst, norm_src, b1.reshape(1, D_H), W2, N_CLASSES)
    p2 = _sc_aggregate(hw2, src_t, dst_t, zeros32, N_CLASSES // 2)
    return _tc_final(p2, norm_dst, b2.reshape(1, N_CLASSES))


# speedup vs baseline: 1.3331x; 1.3331x over previous
import jax
import jax.numpy as jnp
from jax import lax
from jax.experimental import pallas as pl
from jax.experimental.pallas import tpu as pltpu
from jax.experimental.pallas import tpu_sc as plsc

N = 10000
E = 320000
D_IN = 128
D_H = 128
N_CLASSES = 64

NC = 2
NS = 16
NW = NC * NS
EPT = E // NW
CHUNK = 128
NCHUNK = 158
NBUF = 2
E_PAD = NS * NCHUNK * CHUNK
DUMMY = 512
ACC_ROWS = N + DUMMY
ZROWS = 624
RB = 1000

_mesh = plsc.VectorSubcoreMesh(core_axis_name="c", subcore_axis_name="s")
_sc_params = pltpu.CompilerParams(needs_layout_passes=False, use_tc_tiling_on_sc=False)


def _deg_body(src_hbm, dst_hbm, osrc_hbm, odst_hbm, idx_v, hist_v):
    cid = lax.axis_index("c")
    sid = lax.axis_index("s")
    w = cid * NS + sid
    zeros16 = jnp.zeros((16,), jnp.float32)
    ones16 = jnp.ones((16,), jnp.float32)
    for ix_hbm, out_hbm in ((src_hbm, osrc_hbm), (dst_hbm, odst_hbm)):
        pltpu.sync_copy(ix_hbm.at[w], idx_v)

        @pl.loop(0, N // 16)
        def _(i):
            hist_v[pl.ds(i * 16, 16)] = zeros16

        @pl.loop(0, EPT // 16)
        def _(i):
            idx = idx_v[pl.ds(i * 16, 16)]
            plsc.addupdate_scatter(hist_v, [idx], ones16)

        pltpu.sync_copy(hist_v, out_hbm.at[w])


def _sc_degrees(src_flat, dst_flat):
    return pl.kernel(
        _deg_body,
        out_type=(
            jax.ShapeDtypeStruct((NW, N), jnp.float32),
            jax.ShapeDtypeStruct((NW, N), jnp.float32),
        ),
        mesh=_mesh,
        scratch_types=[
            pltpu.VMEM((EPT,), jnp.int32),
            pltpu.VMEM((N,), jnp.float32),
        ],
        compiler_params=_sc_params,
    )(src_flat, dst_flat)


def _make_agg_body(dh):
    def _agg_body(hw_hbm, src_hbm, dst_hbm, zeros_hbm, out_hbm,
                  src_v, dst_v, *rest):
        bufs = rest[:NBUF]
        acc_sh = rest[NBUF]
        gsems = rest[NBUF + 1:2 * NBUF + 1]
        ssems = rest[2 * NBUF + 1:3 * NBUF + 1]
        cid = lax.axis_index("c")
        sid = lax.axis_index("s")
        hw_c = hw_hbm.at[cid]
        pltpu.sync_copy(src_hbm.at[sid], src_v)
        pltpu.sync_copy(dst_hbm.at[sid], dst_v)
        pltpu.sync_copy(zeros_hbm, acc_sh.at[pl.ds(sid * ZROWS, ZROWS)])

        @pl.when(sid == NS - 1)
        def _():
            pltpu.sync_copy(zeros_hbm.at[pl.ds(0, ACC_ROWS - NS * ZROWS)],
                            acc_sh.at[pl.ds(NS * ZROWS, ACC_ROWS - NS * ZROWS)])

        plsc.subcore_barrier()

        buf_a, buf_b = bufs[0], bufs[1]
        sem_a, sem_b = gsems[0], gsems[1]
        pltpu.async_copy(hw_c.at[src_v.at[0]], buf_a, sem_a)

        @pl.loop(0, NCHUNK, step=2)
        def _(j):
            pltpu.async_copy(hw_c.at[src_v.at[j + 1]], buf_b, sem_b)
            pltpu.make_async_copy(hw_c.at[src_v.at[j]], buf_a, sem_a).wait()
            pltpu.sync_copy(buf_a, acc_sh.at[dst_v.at[j]], add=True)

            @pl.when(j + 2 < NCHUNK)
            def _():
                pltpu.async_copy(hw_c.at[src_v.at[j + 2]], buf_a, sem_a)

            pltpu.make_async_copy(hw_c.at[src_v.at[j + 1]], buf_b, sem_b).wait()
            pltpu.sync_copy(buf_b, acc_sh.at[dst_v.at[j + 1]], add=True)

        plsc.subcore_barrier()

        @pl.loop(0, ZROWS // 104)
        def _(k):
            r0 = sid * ZROWS + k * 104
            pltpu.sync_copy(acc_sh.at[pl.ds(r0, 104)], bufs[0].at[pl.ds(0, 104)])
            pltpu.sync_copy(bufs[0].at[pl.ds(0, 104)],
                            out_hbm.at[cid, pl.ds(r0, 104)])

        @pl.when(sid == NS - 1)
        def _():
            r0 = NS * ZROWS
            pltpu.sync_copy(acc_sh.at[pl.ds(r0, N - r0)],
                            bufs[0].at[pl.ds(0, N - r0)])
            pltpu.sync_copy(bufs[0].at[pl.ds(0, N - r0)],
                            out_hbm.at[cid, pl.ds(r0, N - r0)])

    return _agg_body


def _sc_aggregate(hw, src_t, dst_t, zeros_blk, dh):
    return pl.kernel(
        _make_agg_body(dh),
        out_type=jax.ShapeDtypeStruct((NC, N, dh), jnp.float32),
        mesh=_mesh,
        scratch_types=[
            pltpu.VMEM((NCHUNK, CHUNK), jnp.int32),
            pltpu.VMEM((NCHUNK, CHUNK), jnp.int32),
        ] + [pltpu.VMEM((CHUNK, dh), jnp.float32) for _ in range(NBUF)]
        + [pltpu.VMEM_SHARED((ACC_ROWS, dh), jnp.float32)]
        + [pltpu.SemaphoreType.DMA for _ in range(2 * NBUF)],
        compiler_params=_sc_params,
    )(hw, src_t, dst_t, zeros_blk)


def _fused0_body(x_ref, w_ref, hs_ref, hd_ref, ns_ref, nd_ref, hw_ref):
    ns = lax.rsqrt(jnp.clip(jnp.sum(hs_ref[...], axis=0), 1.0, None))
    nd = lax.rsqrt(jnp.clip(jnp.sum(hd_ref[...], axis=0), 1.0, None))
    ns_ref[...] = ns[:, None]
    nd_ref[...] = nd[:, None]
    h = jnp.maximum(x_ref[...], 0.0) * ns[:, None]
    hw = jnp.dot(h, w_ref[...], preferred_element_type=jnp.float32)
    hw_ref[0] = hw[:, : D_H // 2]
    hw_ref[1] = hw[:, D_H // 2:]


def _tc_fused0(x, w0, hist_src, hist_dst):
    return pl.pallas_call(
        _fused0_body,
        out_shape=[
            jax.ShapeDtypeStruct((N, 1), jnp.float32),
            jax.ShapeDtypeStruct((N, 1), jnp.float32),
            jax.ShapeDtypeStruct((NC, N, D_H // 2), jnp.float32),
        ],
    )(x, w0, hist_src, hist_dst)


def _layer_body(p_ref, nd_ref, ns_ref, b_ref, w_ref, o_ref):
    agg = jnp.concatenate([p_ref[0], p_ref[1]], axis=1)
    h = jnp.maximum(agg * nd_ref[...] + b_ref[...], 0.0)
    hw = jnp.dot(h * ns_ref[...], w_ref[...],
                 preferred_element_type=jnp.float32)
    dh = hw.shape[-1] // 2
    o_ref[0] = hw[:, :dh]
    o_ref[1] = hw[:, dh:]


def _tc_layer(p, norm_dst, norm_src, b, w, d_out):
    d_in_h = p.shape[-1]
    return pl.pallas_call(
        _layer_body,
        grid=(N // RB,),
        in_specs=[
            pl.BlockSpec((NC, RB, d_in_h), lambda i: (0, i, 0)),
            pl.BlockSpec((RB, 1), lambda i: (i, 0)),
            pl.BlockSpec((RB, 1), lambda i: (i, 0)),
            pl.BlockSpec((1, 2 * d_in_h), lambda i: (0, 0)),
            pl.BlockSpec((2 * d_in_h, d_out), lambda i: (0, 0)),
        ],
        out_specs=pl.BlockSpec((NC, RB, d_out // 2), lambda i: (0, i, 0)),
        out_shape=jax.ShapeDtypeStruct((NC, N, d_out // 2), jnp.float32),
    )(p, norm_dst, norm_src, b, w)


def _final_body(p_ref, nd_ref, b_ref, o_ref):
    agg = jnp.concatenate([p_ref[0], p_ref[1]], axis=1)
    o_ref[...] = agg * nd_ref[...] + b_ref[...]


def _tc_final(p, norm_dst, b):
    dh = p.shape[-1]
    return pl.pallas_call(
        _final_body,
        grid=(N // RB,),
        in_specs=[
            pl.BlockSpec((NC, RB, dh), lambda i: (0, i, 0)),
            pl.BlockSpec((RB, 1), lambda i: (i, 0)),
            pl.BlockSpec((1, 2 * dh), lambda i: (0, 0)),
        ],
        out_specs=pl.BlockSpec((RB, 2 * dh), lambda i: (i, 0)),
        out_shape=jax.ShapeDtypeStruct((N, 2 * dh), jnp.float32),
    )(p, norm_dst, b)


def kernel(x, edge_index, W0, b0, W1, b1, W2, b2):
    src = edge_index[0]
    dst = edge_index[1]
    src_flat = src.reshape(NW, EPT)
    dst_flat = dst.reshape(NW, EPT)
    pad = E_PAD - E
    src_t = jnp.concatenate(
        [src, jnp.zeros((pad,), jnp.int32)]).reshape(NS, NCHUNK, CHUNK)
    dst_t = jnp.concatenate(
        [dst, N + (jnp.arange(pad, dtype=jnp.int32) % DUMMY)]
    ).reshape(NS, NCHUNK, CHUNK)
    zeros64 = jnp.zeros((ZROWS, D_H // 2), jnp.float32)
    zeros32 = jnp.zeros((ZROWS, N_CLASSES // 2), jnp.float32)

    hist_src, hist_dst = _sc_degrees(src_flat, dst_flat)
    norm_src, norm_dst, hw0 = _tc_fused0(x, W0, hist_src, hist_dst)

    p0 = _sc_aggregate(hw0, src_t, dst_t, zeros64, D_H // 2)
    hw1 = _tc_layer(p0, norm_dst, norm_src, b0.reshape(1, D_H), W1, D_H)
    p1 = _sc_aggregate(hw1, src_t, dst_t, zeros64, D_H // 2)
    hw2 = _tc_layer(p1, norm_dst, norm_src, b1.reshape(1, D_H), W2, N_CLASSES)
    p2 = _sc_aggregate(hw2, src_t, dst_t, zeros32, N_CLASSES // 2)
    return _tc_final(p2, norm_dst, b2.reshape(1, N_CLASSES))
